# baseline (speedup 1.0000x reference)
import functools

import jax
import jax.numpy as jnp
from jax import lax
from jax.experimental import pallas as pl
from jax.experimental.pallas import tpu as pltpu
from jax.experimental.pallas import tpu_sc as plsc

N = 65536
NNZ = 4294967
K = 4096
R = 16384
L = 3

NUM_CORES = 2
NUM_SUB = 16
NUM_TILES = NUM_CORES * NUM_SUB
CHUNK = 5120
CHUNKS_PER_TILE = 26
EDGES_PER_TILE = CHUNK * CHUNKS_PER_TILE
NNZ_MAIN = NUM_TILES * EDGES_PER_TILE
NNZ_TAIL = NNZ - NNZ_MAIN
TAIL_PAD = NUM_TILES * CHUNK
ZSL = N // NUM_SUB

_mesh = plsc.VectorSubcoreMesh(
    core_axis_name="c", subcore_axis_name="s", num_cores=NUM_CORES)


@functools.partial(
    pl.kernel,
    out_type=(jax.ShapeDtypeStruct((NUM_TILES, 16), jnp.float32),
              jax.ShapeDtypeStruct((NUM_CORES, N), jnp.float32)),
    mesh=_mesh,
    scratch_types=[
        pltpu.VMEM((N,), jnp.float32),
        [pltpu.VMEM((CHUNK,), jnp.int32)] * 2,
        [pltpu.VMEM((CHUNK,), jnp.float32)] * 2,
        [pltpu.VMEM((CHUNK,), jnp.float32)] * 2,
        [pltpu.VMEM((CHUNK,), jnp.int32)] * 2,
        pltpu.VMEM((R // NUM_SUB,), jnp.float32),
        pltpu.VMEM((R // NUM_SUB,), jnp.int32),
        pltpu.VMEM((K // NUM_SUB,), jnp.float32),
        pltpu.VMEM((K // NUM_SUB,), jnp.int32),
        pltpu.VMEM((16,), jnp.float32),
        pltpu.VMEM((ZSL,), jnp.float32),
        pltpu.VMEM((ZSL,), jnp.float32),
        pltpu.VMEM((ZSL,), jnp.int32),
        pltpu.VMEM_SHARED((N,), jnp.float32),
        [pltpu.SemaphoreType.DMA] * 2,
        [pltpu.SemaphoreType.DMA] * 2,
        pltpu.SemaphoreType.REGULAR,
    ],
    compiler_params=pltpu.CompilerParams(needs_layout_passes=False),
)
def _sc_model(xs_h, cti_h, g_h, dec_h, col_h, w_h, row_h,
              colt_h, wt_h, rowt_h, out_h, part_h,
              h_old, col_b, w_b, prod_b, row_b, xs_buf, cti_buf,
              g_buf, dec_buf, acc_buf, zero_buf, tmp_buf, idx_buf, h_acc,
              sem_ld, sem_sc, sem_cb):
    cid = lax.axis_index("c")
    sid = lax.axis_index("s")
    gid = cid * NUM_SUB + sid
    zero16 = jnp.zeros((16,), jnp.float32)
    iota16 = lax.broadcasted_iota(jnp.int32, (16,), 0)

    def _z(j, _):
        zero_buf[pl.ds(j * 16, 16)] = zero16
        idx_buf[pl.ds(j * 16, 16)] = sid * ZSL + j * 16 + iota16
        return 0
    lax.fori_loop(0, ZSL // 16, _z, 0)
    pltpu.sync_copy(zero_buf, h_acc.at[pl.ds(sid * ZSL, ZSL)])
    plsc.subcore_barrier()

    rpt = R // NUM_SUB
    pltpu.sync_copy(xs_h.at[pl.ds(sid * rpt, rpt)], xs_buf)
    pltpu.sync_copy(cti_h.at[pl.ds(sid * rpt, rpt)], cti_buf)
    pltpu.sync_copy(xs_buf, h_acc.at[cti_buf], add=True)
    plsc.subcore_barrier()

    def _start_loads(p, off, srcs):
        ch, wh, rh = srcs
        pltpu.async_copy(ch.at[pl.ds(off, CHUNK)], col_b[p], sem_ld[p])
        pltpu.async_copy(wh.at[pl.ds(off, CHUNK)], w_b[p], sem_ld[p])
        pltpu.async_copy(rh.at[pl.ds(off, CHUNK)], row_b[p], sem_ld[p])

    def _wait_loads(p, off, srcs):
        ch, wh, rh = srcs
        pltpu.make_async_copy(ch.at[pl.ds(off, CHUNK)], col_b[p],
                              sem_ld[p]).wait()
        pltpu.make_async_copy(wh.at[pl.ds(off, CHUNK)], w_b[p],
                              sem_ld[p]).wait()
        pltpu.make_async_copy(rh.at[pl.ds(off, CHUNK)], row_b[p],
                              sem_ld[p]).wait()

    def _wait_scatter(p):
        pltpu.make_async_copy(prod_b[p], h_acc.at[row_b[p]],
                              sem_sc[p]).wait()

    main_srcs = (col_h, w_h, row_h)
    tail_srcs = (colt_h, wt_h, rowt_h)
    NCH = CHUNKS_PER_TILE + 1

    def _src_off(i, base):
        if i < CHUNKS_PER_TILE:
            return main_srcs, base + i * CHUNK
        return tail_srcs, gid * CHUNK

    def _step(_s, _c):
        base = gid * EDGES_PER_TILE
        _start_loads(0, base, main_srcs)
        pltpu.sync_copy(h_acc, h_old)
        plsc.subcore_barrier()
        pltpu.sync_copy(zero_buf, h_acc.at[pl.ds(sid * ZSL, ZSL)])
        plsc.subcore_barrier()

        for i in range(NCH):
            p = i % 2
            s_i, o_i = _src_off(i, base)
            _wait_loads(p, o_i, s_i)
            if i >= 1:
                _wait_scatter(1 - p)
            if i + 1 < NCH:
                s_n, o_n = _src_off(i + 1, base)
                _start_loads(1 - p, o_n, s_n)

            @plsc.parallel_loop(0, CHUNK // 16, unroll=8)
            def _gm(j):
                idx = col_b[p][pl.ds(j * 16, 16)]
                hv = plsc.load_gather(h_old, [idx])
                wv = w_b[p][pl.ds(j * 16, 16)]
                prod_b[p][pl.ds(j * 16, 16)] = hv * wv

            pltpu.async_copy(prod_b[p], h_acc.at[row_b[p]], sem_sc[p],
                             add=True)
        _wait_scatter((NCH - 1) % 2)
        plsc.subcore_barrier()

        @pl.when(_s < L - 1)
        def _combine():
            pltpu.sync_copy(h_acc.at[pl.ds(sid * ZSL, ZSL)],
                            part_h.at[cid].at[pl.ds(sid * ZSL, ZSL)])
            plsc.subcore_barrier()
            pltpu.core_barrier(sem_cb, core_axis_name="c")
            pltpu.sync_copy(part_h.at[1 - cid].at[pl.ds(sid * ZSL, ZSL)],
                            tmp_buf)
            pltpu.sync_copy(tmp_buf, h_acc.at[idx_buf], add=True)
            plsc.subcore_barrier()
        return 0
    lax.fori_loop(0, L, _step, 0)

    pltpu.sync_copy(h_acc, h_old)
    kpt = K // NUM_SUB
    pltpu.sync_copy(g_h.at[pl.ds(sid * kpt, kpt)], g_buf)
    pltpu.sync_copy(dec_h.at[pl.ds(sid * kpt, kpt)], dec_buf)

    def _dot(j, acc):
        idx = dec_buf[pl.ds(j * 16, 16)]
        hv = plsc.load_gather(h_old, [idx])
        gv = g_buf[pl.ds(j * 16, 16)]
        return acc + hv * gv
    acc = lax.fori_loop(0, kpt // 16, _dot, zero16)
    acc_buf[...] = acc
    pltpu.sync_copy(acc_buf, out_h.at[gid])


def kernel(x, adj_w, w_ret, b_ret, dec_val, fc_w, fc_b,
           cell_type_indices, adj_row, adj_col, dec_col):
    xs = x[0] * w_ret + b_ret
    cti = cell_type_indices.astype(jnp.int32)
    g = fc_w[0] * dec_val
    dec = dec_col.astype(jnp.int32)
    pad = TAIL_PAD - NNZ_TAIL
    col_i = adj_col.astype(jnp.int32)
    row_i = adj_row.astype(jnp.int32)
    col_t = jnp.concatenate(
        [lax.slice(col_i, (NNZ_MAIN,), (NNZ,)), jnp.zeros((pad,), jnp.int32)])
    w_t = jnp.concatenate(
        [lax.slice(adj_w, (NNZ_MAIN,), (NNZ,)), jnp.zeros((pad,), jnp.float32)])
    row_t = jnp.concatenate(
        [lax.slice(row_i, (NNZ_MAIN,), (NNZ,)), jnp.zeros((pad,), jnp.int32)])
    parts, _ = _sc_model(xs, cti, g, dec, col_i, adj_w, row_i,
                         col_t, w_t, row_t)
    return jnp.sum(parts)[None] + fc_b


# speedup vs baseline: 1.5368x; 1.5368x over previous
import functools

import jax
import jax.numpy as jnp
from jax import lax
from jax.experimental import pallas as pl
from jax.experimental.pallas import tpu as pltpu
from jax.experimental.pallas import tpu_sc as plsc

N = 65536
NNZ = 4294967
K = 4096
R = 16384
L = 3

NUM_CORES = 2
NUM_SUB = 16
NUM_TILES = NUM_CORES * NUM_SUB
CHUNK = 4096
CHUNKS_PER_TILE = 32
EDGES_PER_TILE = CHUNK * CHUNKS_PER_TILE
NNZ_MAIN = NUM_TILES * EDGES_PER_TILE
NNZ_TAIL = NNZ - NNZ_MAIN
TAIL_PAD = NUM_TILES * CHUNK
ZSL = N // NUM_SUB

_mesh = plsc.VectorSubcoreMesh(
    core_axis_name="c", subcore_axis_name="s", num_cores=NUM_CORES)


@functools.partial(
    pl.kernel,
    out_type=(jax.ShapeDtypeStruct((NUM_TILES, 16), jnp.float32),
              jax.ShapeDtypeStruct((NUM_CORES, N), jnp.float32)),
    mesh=_mesh,
    scratch_types=[
        pltpu.VMEM((N,), jnp.float32),
        [pltpu.VMEM((CHUNK,), jnp.int32)] * 2,
        [pltpu.VMEM((CHUNK,), jnp.float32)] * 2,
        [pltpu.VMEM((CHUNK,), jnp.float32)] * 2,
        [pltpu.VMEM((CHUNK,), jnp.int32)] * 2,
        pltpu.VMEM((R // NUM_SUB,), jnp.float32),
        pltpu.VMEM((R // NUM_SUB,), jnp.int32),
        pltpu.VMEM((K // NUM_SUB,), jnp.float32),
        pltpu.VMEM((K // NUM_SUB,), jnp.int32),
        pltpu.VMEM((16,), jnp.float32),
        pltpu.VMEM((ZSL,), jnp.float32),
        pltpu.VMEM((ZSL,), jnp.float32),
        pltpu.VMEM((ZSL,), jnp.int32),
        pltpu.VMEM_SHARED((N,), jnp.float32),
        [pltpu.SemaphoreType.DMA] * 2,
        [pltpu.SemaphoreType.DMA] * 2,
        pltpu.SemaphoreType.REGULAR,
    ],
    compiler_params=pltpu.CompilerParams(needs_layout_passes=False),
)
def _sc_model(xs_h, cti_h, g_h, dec_h, col_h, w_h, row_h,
              colt_h, wt_h, rowt_h, out_h, part_h,
              h_old, col_b, w_b, prod_b, row_b, xs_buf, cti_buf,
              g_buf, dec_buf, acc_buf, zero_buf, tmp_buf, idx_buf, h_acc,
              sem_ld, sem_sc, sem_cb):
    cid = lax.axis_index("c")
    sid = lax.axis_index("s")
    gid = cid * NUM_SUB + sid
    zero16 = jnp.zeros((16,), jnp.float32)
    iota16 = lax.broadcasted_iota(jnp.int32, (16,), 0)

    def _z(j, _):
        zero_buf[pl.ds(j * 16, 16)] = zero16
        idx_buf[pl.ds(j * 16, 16)] = sid * ZSL + j * 16 + iota16
        return 0
    lax.fori_loop(0, ZSL // 16, _z, 0)
    pltpu.sync_copy(zero_buf, h_acc.at[pl.ds(sid * ZSL, ZSL)])
    plsc.subcore_barrier()

    rpt = R // NUM_SUB
    pltpu.sync_copy(xs_h.at[pl.ds(sid * rpt, rpt)], xs_buf)
    pltpu.sync_copy(cti_h.at[pl.ds(sid * rpt, rpt)], cti_buf)
    pltpu.sync_copy(xs_buf, h_acc.at[cti_buf], add=True)
    plsc.subcore_barrier()

    def _start_loads(p, off, srcs):
        ch, wh, rh = srcs
        pltpu.async_copy(ch.at[pl.ds(off, CHUNK)], col_b[p], sem_ld[p])
        pltpu.async_copy(wh.at[pl.ds(off, CHUNK)], w_b[p], sem_ld[p])
        pltpu.async_copy(rh.at[pl.ds(off, CHUNK)], row_b[p], sem_ld[p])

    def _wait_loads(p, off, srcs):
        ch, wh, rh = srcs
        pltpu.make_async_copy(ch.at[pl.ds(off, CHUNK)], col_b[p],
                              sem_ld[p]).wait()
        pltpu.make_async_copy(wh.at[pl.ds(off, CHUNK)], w_b[p],
                              sem_ld[p]).wait()
        pltpu.make_async_copy(rh.at[pl.ds(off, CHUNK)], row_b[p],
                              sem_ld[p]).wait()

    def _wait_scatter(p):
        pltpu.make_async_copy(prod_b[p], h_acc.at[row_b[p]],
                              sem_sc[p]).wait()

    main_srcs = (col_h, w_h, row_h)
    tail_srcs = (colt_h, wt_h, rowt_h)
    NCH = CHUNKS_PER_TILE + 1

    def _src_off(i, base):
        if i < CHUNKS_PER_TILE:
            return main_srcs, base + i * CHUNK
        return tail_srcs, gid * CHUNK

    def _step(_s, _c):
        base = gid * EDGES_PER_TILE
        _start_loads(0, base, main_srcs)
        pltpu.sync_copy(h_acc, h_old)
        plsc.subcore_barrier()
        pltpu.sync_copy(zero_buf, h_acc.at[pl.ds(sid * ZSL, ZSL)])
        plsc.subcore_barrier()

        for i in range(NCH):
            p = i % 2
            s_i, o_i = _src_off(i, base)
            _wait_loads(p, o_i, s_i)
            if i >= 1:
                _wait_scatter(1 - p)
            if i + 1 < NCH:
                s_n, o_n = _src_off(i + 1, base)
                _start_loads(1 - p, o_n, s_n)

            @plsc.parallel_loop(0, CHUNK // 16, unroll=8)
            def _gm(j):
                idx = col_b[p][pl.ds(j * 16, 16)]
                hv = plsc.load_gather(h_old, [idx])
                wv = w_b[p][pl.ds(j * 16, 16)]
                prod_b[p][pl.ds(j * 16, 16)] = hv * wv

            pltpu.async_copy(prod_b[p], h_acc.at[row_b[p]], sem_sc[p],
                             add=True)
        _wait_scatter((NCH - 1) % 2)
        plsc.subcore_barrier()

        @pl.when(_s < L - 1)
        def _combine():
            pltpu.sync_copy(h_acc.at[pl.ds(sid * ZSL, ZSL)],
                            part_h.at[cid].at[pl.ds(sid * ZSL, ZSL)])
            plsc.subcore_barrier()
            pltpu.core_barrier(sem_cb, core_axis_name="c")
            pltpu.sync_copy(part_h.at[1 - cid].at[pl.ds(sid * ZSL, ZSL)],
                            tmp_buf)
            pltpu.sync_copy(tmp_buf, h_acc.at[idx_buf], add=True)
            plsc.subcore_barrier()
        return 0
    lax.fori_loop(0, L, _step, 0)

    pltpu.sync_copy(h_acc, h_old)
    kpt = K // NUM_SUB
    pltpu.sync_copy(g_h.at[pl.ds(sid * kpt, kpt)], g_buf)
    pltpu.sync_copy(dec_h.at[pl.ds(sid * kpt, kpt)], dec_buf)

    def _dot(j, acc):
        idx = dec_buf[pl.ds(j * 16, 16)]
        hv = plsc.load_gather(h_old, [idx])
        gv = g_buf[pl.ds(j * 16, 16)]
        return acc + hv * gv
    acc = lax.fori_loop(0, kpt // 16, _dot, zero16)
    acc_buf[...] = acc
    pltpu.sync_copy(acc_buf, out_h.at[gid])


def kernel(x, adj_w, w_ret, b_ret, dec_val, fc_w, fc_b,
           cell_type_indices, adj_row, adj_col, dec_col):
    xs = x[0] * w_ret + b_ret
    cti = cell_type_indices.astype(jnp.int32)
    g = fc_w[0] * dec_val
    dec = dec_col.astype(jnp.int32)
    pad = TAIL_PAD - NNZ_TAIL
    col_i = adj_col.astype(jnp.int32)
    row_i = adj_row.astype(jnp.int32)
    col_t = jnp.concatenate(
        [lax.slice(col_i, (NNZ_MAIN,), (NNZ,)), jnp.zeros((pad,), jnp.int32)])
    w_t = jnp.concatenate(
        [lax.slice(adj_w, (NNZ_MAIN,), (NNZ,)), jnp.zeros((pad,), jnp.float32)])
    row_t = jnp.concatenate(
        [lax.slice(row_i, (NNZ_MAIN,), (NNZ,)), jnp.zeros((pad,), jnp.int32)])
    parts, _ = _sc_model(xs, cti, g, dec, col_i, adj_w, row_i,
                         col_t, w_t, row_t)
    return jnp.sum(parts)[None] + fc_b
